# Initial kernel scaffold; baseline (speedup 1.0000x reference)
import jax
import jax.numpy as jnp
from jax.experimental import pallas as pl


def kernel(t, betas):
    raise NotImplementedError("write your pallas kernel here")


# speedup vs baseline: 18.8123x; 18.8123x over previous
import functools

import jax
import jax.numpy as jnp
from jax import lax
from jax.experimental import pallas as pl
from jax.experimental.pallas import tpu as pltpu
from jax.experimental.pallas import tpu_sc as plsc

L = 16
T_LEN = 1001
T_PAD = 1008
NVREG = T_PAD // L
NC = 2
NS = 16
NW = NC * NS
NTAB = 6


def _rsqrt(x):
    i = plsc.bitcast(x, jnp.int32)
    y = plsc.bitcast(jnp.int32(0x5F3759DF) - (i >> 1), jnp.float32)
    for _ in range(3):
        y = y * (1.5 - 0.5 * x * y * y)
    return y


def _sqrt(x):
    return jnp.where(x > 0.0, x * _rsqrt(x), 0.0)


def _sc_body(t_hbm, bet_hbm, out_hbm, bet_v, buf_a, buf_b, tab_v, t_v, out_v):
    bpw = t_hbm.shape[0] // NW
    gv = bpw // L
    wid = lax.axis_index("s") * NC + lax.axis_index("c")
    base = wid * bpw

    pltpu.sync_copy(bet_hbm, bet_v)
    pltpu.sync_copy(t_hbm.at[pl.ds(base, bpw)], t_v)

    iota = lax.iota(jnp.int32, L)

    def init_body(i, _):
        s = pl.ds(i * L, L)
        buf_a[s] = 1.0 - bet_v[s]
        return 0

    lax.fori_loop(0, NVREG, init_body, 0)

    def hs_pass(src, dst, k):
        def body(i, _):
            s = pl.ds(i * L, L)
            p = i * L + iota
            x = src[s]
            idx = jnp.maximum(p - k, 0)
            y = plsc.load_gather(src, [idx])
            y = jnp.where(p >= k, y, 1.0)
            dst[s] = x * y
            return 0

        lax.fori_loop(0, NVREG, body, 0)

    src, dst = buf_a, buf_b
    for k in (1, 2, 4, 8, 16, 32, 64, 128, 256, 512):
        hs_pass(src, dst, k)
        src, dst = dst, src

    def deriv_body(i, _):
        s = i * L
        b = bet_v[pl.ds(s, L)]
        ab = buf_a[pl.ds(s, L)]
        tab_v[pl.ds(0 * T_PAD + s, L)] = b
        tab_v[pl.ds(1 * T_PAD + s, L)] = _sqrt(b)
        tab_v[pl.ds(2 * T_PAD + s, L)] = ab
        tab_v[pl.ds(3 * T_PAD + s, L)] = _sqrt(ab)
        tab_v[pl.ds(4 * T_PAD + s, L)] = _sqrt(1.0 - ab)
        tab_v[pl.ds(5 * T_PAD + s, L)] = _rsqrt(1.0 - b)
        return 0

    lax.fori_loop(0, NVREG, deriv_body, 0)

    def gat_body(i, _):
        s = pl.ds(i * L, L)
        idx = t_v[s]
        for j in range(NTAB):
            out_v[j, s] = plsc.load_gather(tab_v, [idx + (j * T_PAD)])
        return 0

    lax.fori_loop(0, gv, gat_body, 0)

    for j in range(NTAB):
        pltpu.sync_copy(out_v.at[j], out_hbm.at[j, pl.ds(base, bpw)])


def _make_sc_call(batch):
    bpw = batch // NW
    mesh = plsc.VectorSubcoreMesh(core_axis_name="c", subcore_axis_name="s")
    return pl.kernel(
        _sc_body,
        mesh=mesh,
        compiler_params=pltpu.CompilerParams(needs_layout_passes=False),
        out_type=jax.ShapeDtypeStruct((NTAB, batch), jnp.float32),
        scratch_types=[
            pltpu.VMEM((T_PAD,), jnp.float32),
            pltpu.VMEM((T_PAD,), jnp.float32),
            pltpu.VMEM((T_PAD,), jnp.float32),
            pltpu.VMEM((NTAB * T_PAD,), jnp.float32),
            pltpu.VMEM((bpw,), jnp.int32),
            pltpu.VMEM((NTAB, bpw), jnp.float32),
        ],
    )


@jax.jit
def kernel(t, betas):
    betas_p = jnp.pad(betas, (0, T_PAD - betas.shape[0]))
    out2d = _make_sc_call(t.shape[0])(t, betas_p)
    return out2d.reshape(NTAB, -1, 1, 1, 1)


# speedup vs baseline: 21.7805x; 1.1578x over previous
import functools

import jax
import jax.numpy as jnp
from jax import lax
from jax.experimental import pallas as pl
from jax.experimental.pallas import tpu as pltpu
from jax.experimental.pallas import tpu_sc as plsc

L = 16
T_LEN = 1001
T_PAD = 1008
NVREG = T_PAD // L
NC = 2
NS = 16
NW = NC * NS
NTAB = 6
NTOTV = (NVREG + L - 1) // L


def _rsqrt(x):
    i = plsc.bitcast(x, jnp.int32)
    y = plsc.bitcast(jnp.int32(0x5F3759DF) - (i >> 1), jnp.float32)
    for _ in range(3):
        y = y * (1.5 - 0.5 * x * y * y)
    return y


def _sqrt(x):
    return jnp.where(x > 0.0, x * _rsqrt(x), 0.0)


def _sc_body(t_hbm, bet_hbm, out_hbm, bet_v, lg_v, ps_v, tot_v, tab_v, t_v,
             out_v, sem_b, sem_t):
    batch = t_hbm.shape[0]
    bpw = batch // NW
    gv = bpw // L
    wid = lax.axis_index("s") * NC + lax.axis_index("c")
    base = wid * bpw

    cp_b = pltpu.async_copy(bet_hbm, bet_v, sem_b)
    cp_t = pltpu.async_copy(t_hbm.at[pl.ds(base, bpw)], t_v, sem_t)
    cp_b.wait()

    iota = lax.iota(jnp.int32, L)

    def p1_body(i, _):
        s = pl.ds(i * L, L)
        b = bet_v[s]
        p = 0.25 + b * 0.2
        p = 1.0 / 3.0 + b * p
        p = 0.5 + b * p
        l = -b * (1.0 + b * p)
        lg_v[s] = l
        ps_v[s] = plsc.cumsum(l)
        return 0

    lax.fori_loop(0, NVREG, p1_body, 0)

    carry = jnp.zeros((L,), jnp.float32)
    for g in range(NTOTV):
        vid = jnp.minimum(g * L + iota, NVREG - 1)
        tg = plsc.load_gather(ps_v, [vid * L + 15])
        sg = plsc.cumsum(tg) + carry
        tot_v[pl.ds(g * L, L)] = sg
        carry = plsc.load_gather(
            tot_v, [jnp.zeros((L,), jnp.int32) + (g * L + 15)]
        )

    def p2_body(i, _):
        s = i * L
        sl = pl.ds(s, L)
        b = bet_v[sl]
        l = lg_v[sl]
        e_idx = jnp.zeros((L,), jnp.int32) + jnp.maximum(i - 1, 0)
        e = plsc.load_gather(tot_v, [e_idx])
        e = jnp.where(i >= 1, e, 0.0)
        big_s = ps_v[sl] + e
        ab = jnp.exp(big_s)
        tab_v[pl.ds(0 * T_PAD + s, L)] = b
        tab_v[pl.ds(1 * T_PAD + s, L)] = _sqrt(b)
        tab_v[pl.ds(2 * T_PAD + s, L)] = ab
        tab_v[pl.ds(3 * T_PAD + s, L)] = jnp.exp(0.5 * big_s)
        tab_v[pl.ds(4 * T_PAD + s, L)] = _sqrt(1.0 - ab)
        tab_v[pl.ds(5 * T_PAD + s, L)] = jnp.exp(-0.5 * l)
        return 0

    lax.fori_loop(0, NVREG, p2_body, 0)

    cp_t.wait()

    def gat_body(i, _):
        sl = pl.ds(i * L, L)
        idx = t_v[sl]
        for j in range(NTAB):
            out_v[j, sl] = plsc.load_gather(tab_v, [idx + (j * T_PAD)])
        return 0

    lax.fori_loop(0, gv, gat_body, 0)

    for j in range(NTAB):
        pltpu.sync_copy(out_v.at[j], out_hbm.at[j, pl.ds(base, bpw)])


def _make_sc_call(batch):
    bpw = batch // NW
    mesh = plsc.VectorSubcoreMesh(core_axis_name="c", subcore_axis_name="s")
    return pl.kernel(
        _sc_body,
        mesh=mesh,
        compiler_params=pltpu.CompilerParams(needs_layout_passes=False),
        out_type=jax.ShapeDtypeStruct((NTAB, batch), jnp.float32),
        scratch_types=[
            pltpu.VMEM((T_PAD,), jnp.float32),
            pltpu.VMEM((T_PAD,), jnp.float32),
            pltpu.VMEM((T_PAD,), jnp.float32),
            pltpu.VMEM((NTOTV * L,), jnp.float32),
            pltpu.VMEM((NTAB * T_PAD,), jnp.float32),
            pltpu.VMEM((bpw,), jnp.int32),
            pltpu.VMEM((NTAB, bpw), jnp.float32),
            pltpu.SemaphoreType.DMA,
            pltpu.SemaphoreType.DMA,
        ],
    )


@jax.jit
def kernel(t, betas):
    betas_p = jnp.pad(betas, (0, T_PAD - betas.shape[0]))
    out2d = _make_sc_call(t.shape[0])(t, betas_p)
    return out2d.reshape(NTAB, -1, 1, 1, 1)


# speedup vs baseline: 25.5684x; 1.1739x over previous
import functools

import jax
import jax.numpy as jnp
from jax import lax
from jax.experimental import pallas as pl
from jax.experimental.pallas import tpu as pltpu
from jax.experimental.pallas import tpu_sc as plsc

L = 16
T_LEN = 1001
T_PAD = 1008
NVREG = T_PAD // L
NC = 2
NS = 16
NW = NC * NS
NTAB = 6
NTOTV = (NVREG + L - 1) // L


def _rsqrt(x):
    i = plsc.bitcast(x, jnp.int32)
    y = plsc.bitcast(jnp.int32(0x5F3759DF) - (i >> 1), jnp.float32)
    for _ in range(3):
        y = y * (1.5 - 0.5 * x * y * y)
    return y


def _sqrt(x):
    return jnp.where(x > 0.0, x * _rsqrt(x), 0.0)


def _sc_body(t_hbm, bet_hbm, out_hbm, bet_v, lg_v, ps_v, tot_v, tab_v, t_v,
             out_v, sem_b, sem_t):
    batch = t_hbm.shape[0]
    bpw = batch // NW
    gv = bpw // L
    wid = lax.axis_index("s") * NC + lax.axis_index("c")
    base = wid * bpw

    bet_v[pl.ds(T_PAD - L, L)] = jnp.zeros((L,), jnp.float32)
    cp_b = pltpu.async_copy(bet_hbm, bet_v.at[pl.ds(0, T_LEN)], sem_b)
    cp_t = pltpu.async_copy(t_hbm.at[pl.ds(base, bpw)], t_v, sem_t)
    cp_b.wait()

    iota = lax.iota(jnp.int32, L)

    @plsc.parallel_loop(0, NVREG, unroll=3)
    def p1_body(i):
        s = pl.ds(i * L, L)
        b = bet_v[s]
        p = 0.25 + b * 0.2
        p = 1.0 / 3.0 + b * p
        p = 0.5 + b * p
        l = -b * (1.0 + b * p)
        lg_v[s] = l
        ps_v[s] = plsc.cumsum(l)

    carry = jnp.zeros((L,), jnp.float32)
    for g in range(NTOTV):
        vid = jnp.minimum(g * L + iota, NVREG - 1)
        tg = plsc.load_gather(ps_v, [vid * L + 15])
        sg = plsc.cumsum(tg) + carry
        tot_v[pl.ds(g * L, L)] = sg
        carry = plsc.load_gather(
            tot_v, [jnp.zeros((L,), jnp.int32) + (g * L + 15)]
        )

    @plsc.parallel_loop(0, NVREG, unroll=2)
    def p2_body(i):
        s = i * L
        sl = pl.ds(s, L)
        b = bet_v[sl]
        l = lg_v[sl]
        e_idx = jnp.zeros((L,), jnp.int32) + jnp.maximum(i - 1, 0)
        e = plsc.load_gather(tot_v, [e_idx])
        e = jnp.where(i >= 1, e, 0.0)
        big_s = ps_v[sl] + e
        ab = jnp.exp(big_s)
        tab_v[pl.ds(0 * T_PAD + s, L)] = b
        tab_v[pl.ds(1 * T_PAD + s, L)] = _sqrt(b)
        tab_v[pl.ds(2 * T_PAD + s, L)] = ab
        tab_v[pl.ds(3 * T_PAD + s, L)] = jnp.exp(0.5 * big_s)
        tab_v[pl.ds(4 * T_PAD + s, L)] = _sqrt(1.0 - ab)
        tab_v[pl.ds(5 * T_PAD + s, L)] = jnp.exp(-0.5 * l)

    cp_t.wait()

    @plsc.parallel_loop(0, gv, unroll=2)
    def gat_body(i):
        sl = pl.ds(i * L, L)
        idx = t_v[sl]
        for j in range(NTAB):
            out_v[j, sl] = plsc.load_gather(tab_v, [idx + (j * T_PAD)])

    for j in range(NTAB):
        pltpu.sync_copy(
            out_v.at[j], out_hbm.at[pl.ds(j * batch + base, bpw)]
        )


def _make_sc_call(batch):
    bpw = batch // NW
    mesh = plsc.VectorSubcoreMesh(core_axis_name="c", subcore_axis_name="s")
    return pl.kernel(
        _sc_body,
        mesh=mesh,
        compiler_params=pltpu.CompilerParams(needs_layout_passes=False),
        out_type=jax.ShapeDtypeStruct((NTAB * batch,), jnp.float32),
        scratch_types=[
            pltpu.VMEM((T_PAD,), jnp.float32),
            pltpu.VMEM((T_PAD,), jnp.float32),
            pltpu.VMEM((T_PAD,), jnp.float32),
            pltpu.VMEM((NTOTV * L,), jnp.float32),
            pltpu.VMEM((NTAB * T_PAD,), jnp.float32),
            pltpu.VMEM((bpw,), jnp.int32),
            pltpu.VMEM((NTAB, bpw), jnp.float32),
            pltpu.SemaphoreType.DMA,
            pltpu.SemaphoreType.DMA,
        ],
    )


@jax.jit
def kernel(t, betas):
    out1d = _make_sc_call(t.shape[0])(t, betas)
    return out1d.reshape(NTAB, -1, 1, 1, 1)


# speedup vs baseline: 26.9542x; 1.0542x over previous
import functools

import jax
import jax.numpy as jnp
from jax import lax
from jax.experimental import pallas as pl
from jax.experimental.pallas import tpu as pltpu
from jax.experimental.pallas import tpu_sc as plsc

L = 16
T_LEN = 1001
T_PAD = 1008
NVREG = T_PAD // L
NC = 1
NS = 16
NW = NC * NS
NTAB = 6
NTOTV = (NVREG + L - 1) // L


def _rsqrt(x):
    i = plsc.bitcast(x, jnp.int32)
    y = plsc.bitcast(jnp.int32(0x5F3759DF) - (i >> 1), jnp.float32)
    for _ in range(3):
        y = y * (1.5 - 0.5 * x * y * y)
    return y


def _sqrt(x):
    return jnp.where(x > 0.0, x * _rsqrt(x), 0.0)


def _sc_body(t_hbm, bet_hbm, out_hbm, bet_v, lg_v, ps_v, tot_v, tab_v, t_v,
             out_v, sem_b, sem_t):
    batch = t_hbm.shape[0]
    bpw = batch // NW
    gv = bpw // L
    wid = lax.axis_index("s") * NC + lax.axis_index("c")
    base = wid * bpw

    bet_v[pl.ds(T_PAD - L, L)] = jnp.zeros((L,), jnp.float32)
    cp_b = pltpu.async_copy(bet_hbm, bet_v.at[pl.ds(0, T_LEN)], sem_b)
    cp_t = pltpu.async_copy(t_hbm.at[pl.ds(base, bpw)], t_v, sem_t)
    cp_b.wait()

    iota = lax.iota(jnp.int32, L)

    @plsc.parallel_loop(0, NVREG, unroll=3)
    def p1_body(i):
        s = pl.ds(i * L, L)
        b = bet_v[s]
        p = 0.25 + b * 0.2
        p = 1.0 / 3.0 + b * p
        p = 0.5 + b * p
        l = -b * (1.0 + b * p)
        lg_v[s] = l
        ps_v[s] = plsc.cumsum(l)

    carry = jnp.zeros((L,), jnp.float32)
    for g in range(NTOTV):
        vid = jnp.minimum(g * L + iota, NVREG - 1)
        tg = plsc.load_gather(ps_v, [vid * L + 15])
        sg = plsc.cumsum(tg) + carry
        tot_v[pl.ds(g * L, L)] = sg
        carry = plsc.load_gather(
            tot_v, [jnp.zeros((L,), jnp.int32) + (g * L + 15)]
        )

    @plsc.parallel_loop(0, NVREG, unroll=2)
    def p2_body(i):
        s = i * L
        sl = pl.ds(s, L)
        b = bet_v[sl]
        l = lg_v[sl]
        e_idx = jnp.zeros((L,), jnp.int32) + jnp.maximum(i - 1, 0)
        e = plsc.load_gather(tot_v, [e_idx])
        e = jnp.where(i >= 1, e, 0.0)
        big_s = ps_v[sl] + e
        ab = jnp.exp(big_s)
        tab_v[pl.ds(0 * T_PAD + s, L)] = b
        tab_v[pl.ds(1 * T_PAD + s, L)] = _sqrt(b)
        tab_v[pl.ds(2 * T_PAD + s, L)] = ab
        tab_v[pl.ds(3 * T_PAD + s, L)] = jnp.exp(0.5 * big_s)
        tab_v[pl.ds(4 * T_PAD + s, L)] = _sqrt(1.0 - ab)
        tab_v[pl.ds(5 * T_PAD + s, L)] = jnp.exp(-0.5 * l)

    cp_t.wait()

    @plsc.parallel_loop(0, gv, unroll=2)
    def gat_body(i):
        sl = pl.ds(i * L, L)
        idx = t_v[sl]
        for j in range(NTAB):
            out_v[j, sl] = plsc.load_gather(tab_v, [idx + (j * T_PAD)])

    for j in range(NTAB):
        pltpu.sync_copy(
            out_v.at[j], out_hbm.at[pl.ds(j * batch + base, bpw)]
        )


def _make_sc_call(batch):
    bpw = batch // NW
    mesh = plsc.VectorSubcoreMesh(
        core_axis_name="c", subcore_axis_name="s", num_cores=NC
    )
    return pl.kernel(
        _sc_body,
        mesh=mesh,
        compiler_params=pltpu.CompilerParams(needs_layout_passes=False),
        out_type=jax.ShapeDtypeStruct((NTAB * batch,), jnp.float32),
        scratch_types=[
            pltpu.VMEM((T_PAD,), jnp.float32),
            pltpu.VMEM((T_PAD,), jnp.float32),
            pltpu.VMEM((T_PAD,), jnp.float32),
            pltpu.VMEM((NTOTV * L,), jnp.float32),
            pltpu.VMEM((NTAB * T_PAD,), jnp.float32),
            pltpu.VMEM((bpw,), jnp.int32),
            pltpu.VMEM((NTAB, bpw), jnp.float32),
            pltpu.SemaphoreType.DMA,
            pltpu.SemaphoreType.DMA,
        ],
    )


@jax.jit
def kernel(t, betas):
    out1d = _make_sc_call(t.shape[0])(t, betas)
    return out1d.reshape(NTAB, -1, 1, 1, 1)
